# baseline (speedup 1.0000x reference)
import functools

import jax
import jax.numpy as jnp
from jax import lax
from jax.experimental import pallas as pl
from jax.experimental.pallas import tpu as pltpu
from jax.experimental.pallas import tpu_sc as plsc

_CHUNK = 32
_RPT = 128
_SCHUNK = 272


@functools.partial(jax.jit, static_argnums=(1, 2))
def _sc_broadcast_rows(W, batch, seq_len):
    d = W.shape[1]
    info = plsc.get_sparse_core_info()
    nc, ns = info.num_cores, info.num_subcores
    nw = nc * ns

    n_tile_workers = nw - nc
    rows_tile = n_tile_workers * _RPT
    rows_spmem = seq_len - rows_tile
    rows_sc = rows_spmem // nc
    n_chunks = _RPT // _CHUNK
    n_schunks = rows_sc // _SCHUNK
    assert rows_spmem > 0 and rows_sc % _SCHUNK == 0 and _RPT % _CHUNK == 0

    mesh = plsc.VectorSubcoreMesh(core_axis_name="c", subcore_axis_name="s")

    @functools.partial(
        pl.kernel,
        mesh=mesh,
        out_type=jax.ShapeDtypeStruct((batch * seq_len, d), jnp.float32),
        scratch_types=[
            pltpu.VMEM((_CHUNK, d), jnp.float32),
            pltpu.VMEM((_CHUNK, d), jnp.float32),
            pltpu.VMEM_SHARED((_SCHUNK, d), jnp.float32),
            pltpu.VMEM_SHARED((_SCHUNK, d), jnp.float32),
            pltpu.SemaphoreType.DMA,
            pltpu.SemaphoreType.DMA,
            pltpu.SemaphoreType.DMA,
            pltpu.SemaphoreType.DMA,
        ],
    )
    def k(w_hbm, out_hbm, buf0, buf1, sbuf0, sbuf1, in_sem, out_sem, s_in_sem,
          s_out_sem):
        cid = lax.axis_index("c")
        sid = lax.axis_index("s")
        wid = sid * nc + cid

        @pl.when(sid == 0)
        def _spmem_path():
            sbufs = (sbuf0, sbuf1)
            sbase = cid * rows_sc
            pltpu.async_copy(w_hbm.at[pl.ds(sbase, _SCHUNK)], sbuf0, s_in_sem)
            for i in range(n_schunks):
                cur = sbufs[i % 2]
                r0 = sbase + i * _SCHUNK
                pltpu.make_async_copy(
                    w_hbm.at[pl.ds(r0, _SCHUNK)], cur, s_in_sem
                ).wait()
                if i + 1 < n_schunks:
                    pltpu.async_copy(
                        w_hbm.at[pl.ds(r0 + _SCHUNK, _SCHUNK)],
                        sbufs[(i + 1) % 2],
                        s_in_sem,
                    )
                for b in range(batch):
                    pltpu.async_copy(
                        cur, out_hbm.at[pl.ds(b * seq_len + r0, _SCHUNK)],
                        s_out_sem,
                    )
                for b in range(batch):
                    pltpu.make_async_copy(
                        cur, out_hbm.at[pl.ds(b * seq_len + r0, _SCHUNK)],
                        s_out_sem,
                    ).wait()

        @pl.when(sid != 0)
        def _tile_path():
            bufs = (buf0, buf1)
            base = rows_spmem + (wid - nc) * _RPT
            pltpu.async_copy(w_hbm.at[pl.ds(base, _CHUNK)], buf0, in_sem)
            for i in range(n_chunks):
                cur = bufs[i % 2]
                r0 = base + i * _CHUNK
                pltpu.make_async_copy(
                    w_hbm.at[pl.ds(r0, _CHUNK)], cur, in_sem
                ).wait()
                if i + 1 < n_chunks:
                    pltpu.async_copy(
                        w_hbm.at[pl.ds(r0 + _CHUNK, _CHUNK)],
                        bufs[(i + 1) % 2],
                        in_sem,
                    )
                for b in range(batch):
                    pltpu.async_copy(
                        cur, out_hbm.at[pl.ds(b * seq_len + r0, _CHUNK)],
                        out_sem,
                    )
                for b in range(batch):
                    pltpu.make_async_copy(
                        cur, out_hbm.at[pl.ds(b * seq_len + r0, _CHUNK)],
                        out_sem,
                    ).wait()

    return k(W)


def kernel(x, W):
    batch, seq_len = x.shape
    d = W.shape[1]
    flat = _sc_broadcast_rows(W, batch, seq_len)
    return flat.reshape(batch, seq_len, d)


# speedup vs baseline: 1.0823x; 1.0823x over previous
import functools

import jax
import jax.numpy as jnp
from jax import lax
from jax.experimental import pallas as pl
from jax.experimental.pallas import tpu as pltpu
from jax.experimental.pallas import tpu_sc as plsc

_CHUNK = 32


@functools.partial(jax.jit, static_argnums=(1, 2))
def _sc_broadcast_rows(W, batch, seq_len):
    d = W.shape[1]
    info = plsc.get_sparse_core_info()
    nw = info.num_cores * info.num_subcores
    rows_per_w = seq_len // nw
    chunk = min(_CHUNK, rows_per_w)
    n_chunks = rows_per_w // chunk
    mesh = plsc.VectorSubcoreMesh(core_axis_name="c", subcore_axis_name="s")

    @functools.partial(
        pl.kernel,
        mesh=mesh,
        out_type=jax.ShapeDtypeStruct((batch * seq_len, d), jnp.float32),
        scratch_types=[
            pltpu.VMEM((chunk, d), jnp.float32),
            pltpu.VMEM((chunk, d), jnp.float32),
            pltpu.SemaphoreType.DMA,
            pltpu.SemaphoreType.DMA,
        ],
    )
    def k(w_hbm, out_hbm, buf0, buf1, in_sem, out_sem):
        wid = lax.axis_index("s") * info.num_cores + lax.axis_index("c")
        base = wid * rows_per_w
        bufs = (buf0, buf1)

        pltpu.async_copy(w_hbm.at[pl.ds(base, chunk)], buf0, in_sem)

        for i in range(n_chunks):
            cur = bufs[i % 2]
            pltpu.make_async_copy(
                w_hbm.at[pl.ds(base + i * chunk, chunk)], cur, in_sem
            ).wait()
            if i + 1 < n_chunks:
                pltpu.async_copy(
                    w_hbm.at[pl.ds(base + (i + 1) * chunk, chunk)],
                    bufs[(i + 1) % 2],
                    in_sem,
                )
            r0 = base + i * chunk
            for b in range(batch):
                pltpu.async_copy(
                    cur, out_hbm.at[pl.ds(b * seq_len + r0, chunk)], out_sem
                )
            for b in range(batch):
                pltpu.make_async_copy(
                    cur, out_hbm.at[pl.ds(b * seq_len + r0, chunk)], out_sem
                ).wait()

    return k(W)


def kernel(x, W):
    batch, seq_len = x.shape
    d = W.shape[1]
    flat = _sc_broadcast_rows(W, batch, seq_len)
    return flat.reshape(batch, seq_len, d)


# speedup vs baseline: 1.1172x; 1.0322x over previous
import functools

import jax
import jax.numpy as jnp
from jax import lax
from jax.experimental import pallas as pl
from jax.experimental.pallas import tpu as pltpu
from jax.experimental.pallas import tpu_sc as plsc

_CHUNK = 64


@functools.partial(jax.jit, static_argnums=(1, 2))
def _sc_broadcast_rows(W, batch, seq_len):
    d = W.shape[1]
    info = plsc.get_sparse_core_info()
    nw = info.num_cores * info.num_subcores
    rows_per_w = seq_len // nw
    chunk = min(_CHUNK, rows_per_w)
    n_chunks = rows_per_w // chunk
    mesh = plsc.VectorSubcoreMesh(core_axis_name="c", subcore_axis_name="s")

    @functools.partial(
        pl.kernel,
        mesh=mesh,
        out_type=jax.ShapeDtypeStruct((batch * seq_len, d), jnp.float32),
        scratch_types=[
            pltpu.VMEM((chunk, d), jnp.float32),
            pltpu.VMEM((chunk, d), jnp.float32),
            pltpu.SemaphoreType.DMA,
            pltpu.SemaphoreType.DMA,
            pltpu.SemaphoreType.DMA,
        ],
    )
    def k(w_hbm, out_hbm, buf0, buf1, in_sem, out_sem0, out_sem1):
        wid = lax.axis_index("s") * info.num_cores + lax.axis_index("c")
        base = wid * rows_per_w
        bufs = (buf0, buf1)
        osems = (out_sem0, out_sem1)

        def write(i, b):
            src = bufs[i % 2]
            dst = out_hbm.at[pl.ds(b * seq_len + base + i * chunk, chunk)]
            return pltpu.make_async_copy(src, dst, osems[i % 2])

        pltpu.async_copy(w_hbm.at[pl.ds(base, chunk)], buf0, in_sem)

        for i in range(n_chunks):
            cur = bufs[i % 2]
            pltpu.make_async_copy(
                w_hbm.at[pl.ds(base + i * chunk, chunk)], cur, in_sem
            ).wait()
            for b in range(batch):
                write(i, b).start()
            if i >= 1:
                for b in range(batch):
                    write(i - 1, b).wait()
            if i + 1 < n_chunks:
                pltpu.async_copy(
                    w_hbm.at[pl.ds(base + (i + 1) * chunk, chunk)],
                    bufs[(i + 1) % 2],
                    in_sem,
                )
        for b in range(batch):
            write(n_chunks - 1, b).wait()

    return k(W)


def kernel(x, W):
    batch, seq_len = x.shape
    d = W.shape[1]
    flat = _sc_broadcast_rows(W, batch, seq_len)
    return flat.reshape(batch, seq_len, d)


# speedup vs baseline: 1.1319x; 1.0131x over previous
import functools

import jax
import jax.numpy as jnp
from jax import lax
from jax.experimental import pallas as pl
from jax.experimental.pallas import tpu as pltpu
from jax.experimental.pallas import tpu_sc as plsc

_CHUNK = 64


@functools.partial(jax.jit, static_argnums=(1, 2))
def _sc_broadcast_rows(W, batch, seq_len):
    d = W.shape[1]
    info = plsc.get_sparse_core_info()
    nw = info.num_cores * info.num_subcores
    rows_per_w = seq_len // nw
    chunk = min(_CHUNK, rows_per_w)
    n_chunks = rows_per_w // chunk
    mesh = plsc.VectorSubcoreMesh(core_axis_name="c", subcore_axis_name="s")

    @functools.partial(
        pl.kernel,
        mesh=mesh,
        out_type=jax.ShapeDtypeStruct((batch * seq_len, d), jnp.float32),
        scratch_types=[
            pltpu.VMEM((chunk, d), jnp.float32),
            pltpu.VMEM((chunk, d), jnp.float32),
            pltpu.SemaphoreType.DMA,
            pltpu.SemaphoreType.DMA,
        ],
    )
    def k(w_hbm, out_hbm, buf0, buf1, in_sem, out_sem):
        wid = lax.axis_index("s") * info.num_cores + lax.axis_index("c")
        base = wid * rows_per_w
        bufs = (buf0, buf1)

        pltpu.async_copy(w_hbm.at[pl.ds(base, chunk)], buf0, in_sem)

        for i in range(n_chunks):
            cur = bufs[i % 2]
            pltpu.make_async_copy(
                w_hbm.at[pl.ds(base + i * chunk, chunk)], cur, in_sem
            ).wait()
            r0 = base + i * chunk
            for b in range(batch):
                pltpu.async_copy(
                    cur, out_hbm.at[pl.ds(b * seq_len + r0, chunk)], out_sem
                )
            if i + 1 < n_chunks:
                pltpu.async_copy(
                    w_hbm.at[pl.ds(base + (i + 1) * chunk, chunk)],
                    bufs[(i + 1) % 2],
                    in_sem,
                )
            for b in range(batch):
                pltpu.make_async_copy(
                    cur, out_hbm.at[pl.ds(b * seq_len + r0, chunk)], out_sem
                ).wait()

    return k(W)


def kernel(x, W):
    batch, seq_len = x.shape
    d = W.shape[1]
    flat = _sc_broadcast_rows(W, batch, seq_len)
    return flat.reshape(batch, seq_len, d)
